# baseline (speedup 1.0000x reference)
import functools

import jax
import jax.numpy as jnp
from jax import lax
from jax.experimental import pallas as pl
from jax.experimental.pallas import tpu as pltpu
from jax.experimental.pallas import tpu_sc as plsc

N = 50000
E = 800000
G = 64

NC = 2
NS = 16
NW = NC * NS
NP = 51200
RPT = NP // NS
EP = 819200
EPW = EP // NW
CH = 3200
NCH = EPW // CH

BN = 1024
BE = 16384

_f32 = jnp.float32


def _enc_node_body(x_ref, w1, b1, w2, b2, ws, nt_ref, t0_ref, t1_ref):
    h = jnp.maximum(jnp.dot(x_ref[...], w1[...],
                            preferred_element_type=_f32) + b1[...], 0.0)
    nnt = jnp.maximum(
        lax.dot_general(w2[...], h, (((0,), (1,)), ((), ())),
                        preferred_element_type=_f32) + b2[...], 0.0)
    nt_ref[...] = nnt
    ns2t = lax.dot_general(ws[...], nnt, (((0,), (0,)), ((), ())),
                           preferred_element_type=_f32)
    t0_ref[...] = ns2t[0]
    t1_ref[...] = ns2t[1]


def _enc_edge_body(a0_ref, a1_ref, w1, b1, w2, b2, we, ble, e0_ref, e1_ref):
    a0 = a0_ref[...]
    a1 = a1_ref[...]
    h1 = [jnp.maximum(a0 * w1[0, j] + a1 * w1[1, j] + b1[0, j], 0.0)
          for j in range(4)]
    h2 = [jnp.maximum(h1[0] * w2[0, k] + h1[1] * w2[1, k]
                      + h1[2] * w2[2, k] + h1[3] * w2[3, k] + b2[0, k], 0.0)
          for k in range(16)]
    for c, ref in ((0, e0_ref), (1, e1_ref)):
        acc = h2[0] * we[0, c]
        for k in range(1, 16):
            acc = acc + h2[k] * we[k, c]
        ref[...] = acc + ble[0, c]


def _full(shape):
    nd = len(shape)
    return pl.BlockSpec(shape, lambda i: (0,) * nd)


def _sc_body(send_hbm, recv_hbm, ec0_hbm, ec1_hbm, t0_hbm, t1_hbm,
             zeros_hbm, ones_hbm, acc_hbm,
             s_v, r_v, e0_v, e1_v, c0_v, c1_v, ones_v, stage_v,
             sem_l, sem_g, sem_s,
             t0, t1, ai0, ai1, ci, ao0, ao1, co):
    core = lax.axis_index("c")
    sid = lax.axis_index("s")
    wid = sid * NC + core
    r0 = sid * RPT

    pltpu.sync_copy(t0_hbm.at[pl.ds(r0, RPT)], stage_v)
    pltpu.sync_copy(stage_v, t0.at[pl.ds(r0, RPT)])
    pltpu.sync_copy(t1_hbm.at[pl.ds(r0, RPT)], stage_v)
    pltpu.sync_copy(stage_v, t1.at[pl.ds(r0, RPT)])
    pltpu.sync_copy(zeros_hbm.at[pl.ds(r0, RPT)], stage_v)
    for acc in (ai0, ai1, ci, ao0, ao1, co):
        pltpu.sync_copy(stage_v, acc.at[pl.ds(r0, RPT)])
    pltpu.sync_copy(ones_hbm, ones_v)
    plsc.subcore_barrier()

    ebase = wid * EPW

    def chunk(j, carry):
        off = ebase + j * CH
        ld_s = pltpu.async_copy(send_hbm.at[pl.ds(off, CH)], s_v, sem_l)
        ld_r = pltpu.async_copy(recv_hbm.at[pl.ds(off, CH)], r_v, sem_l)
        ld_0 = pltpu.async_copy(ec0_hbm.at[pl.ds(off, CH)], e0_v, sem_l)
        ld_1 = pltpu.async_copy(ec1_hbm.at[pl.ds(off, CH)], e1_v, sem_l)
        ld_s.wait()
        g_0 = pltpu.async_copy(t0.at[s_v], c0_v, sem_g)
        g_1 = pltpu.async_copy(t1.at[s_v], c1_v, sem_g)
        ld_0.wait()
        ld_1.wait()
        g_0.wait()
        g_1.wait()

        def vloop(m, c):
            sl = pl.ds(16 * m, 16)
            e0_v[sl] = jnp.maximum(e0_v[sl] + c0_v[sl], 0.0)
            e1_v[sl] = jnp.maximum(e1_v[sl] + c1_v[sl], 0.0)
            return c

        lax.fori_loop(0, CH // 16, vloop, 0, unroll=False)
        ld_r.wait()

        sc = [pltpu.async_copy(e0_v, ai0.at[r_v], sem_s, add=True),
              pltpu.async_copy(e1_v, ai1.at[r_v], sem_s, add=True),
              pltpu.async_copy(ones_v, ci.at[r_v], sem_s, add=True),
              pltpu.async_copy(e0_v, ao0.at[s_v], sem_s, add=True),
              pltpu.async_copy(e1_v, ao1.at[s_v], sem_s, add=True),
              pltpu.async_copy(ones_v, co.at[s_v], sem_s, add=True)]
        for d in sc:
            d.wait()
        return carry

    lax.fori_loop(0, NCH, chunk, 0, unroll=False)
    plsc.subcore_barrier()

    for arr, acc in enumerate((ai0, ai1, ci, ao0, ao1, co)):
        pltpu.sync_copy(acc.at[pl.ds(r0, RPT)], stage_v)
        pltpu.sync_copy(stage_v,
                        acc_hbm.at[pl.ds(core * (6 * NP) + arr * NP + r0,
                                         RPT)])


_sc_edge_phase = functools.partial(
    pl.kernel,
    out_type=jax.ShapeDtypeStruct((NC * 6 * NP,), _f32),
    mesh=plsc.VectorSubcoreMesh(core_axis_name="c", subcore_axis_name="s",
                                num_cores=NC, num_subcores=NS),
    scratch_types=[
        pltpu.VMEM((CH,), jnp.int32),
        pltpu.VMEM((CH,), jnp.int32),
        pltpu.VMEM((CH,), _f32),
        pltpu.VMEM((CH,), _f32),
        pltpu.VMEM((CH,), _f32),
        pltpu.VMEM((CH,), _f32),
        pltpu.VMEM((CH,), _f32),
        pltpu.VMEM((RPT,), _f32),
        pltpu.SemaphoreType.DMA,
        pltpu.SemaphoreType.DMA,
        pltpu.SemaphoreType.DMA,
        pltpu.VMEM_SHARED((NP,), _f32),
        pltpu.VMEM_SHARED((NP,), _f32),
        pltpu.VMEM_SHARED((NP,), _f32),
        pltpu.VMEM_SHARED((NP,), _f32),
        pltpu.VMEM_SHARED((NP,), _f32),
        pltpu.VMEM_SHARED((NP,), _f32),
        pltpu.VMEM_SHARED((NP,), _f32),
        pltpu.VMEM_SHARED((NP,), _f32),
    ],
)(_sc_body)


def _finalize_body(nt_ref, acc_ref, ng_ref,
                   wnn, wnin, bln, wrn, brn, wge, wgn, blg, wgg, wgnr, brg,
                   nout_ref, gout_ref, s_ref):
    i = pl.program_id(0)
    a12 = acc_ref[...]
    a = a12[0:6] + a12[6:12]
    iagg = a[0:2] / jnp.maximum(a[2:3], 1.0)
    n4 = jnp.maximum(
        lax.dot_general(nt_ref[...], wnn[...], (((0,), (0,)), ((), ())),
                        preferred_element_type=_f32)
        + lax.dot_general(iagg, wnin[...], (((0,), (0,)), ((), ())),
                          preferred_element_type=_f32)
        + bln[...], 0.0)
    no = 1.0 / (1.0 + jnp.exp(-(jnp.dot(n4, wrn[...],
                                        preferred_element_type=_f32)
                                + brn[...])))
    nout_ref[...] = no

    ids = ng_ref[0]
    oh = (lax.broadcasted_iota(jnp.int32, (G, BN), 0) == ids).astype(_f32)
    xx = jnp.concatenate([n4, no, jnp.ones((BN, 1), _f32)], axis=1)
    c_a = jnp.dot(oh, xx, preferred_element_type=_f32)
    c_b = lax.dot_general(oh, a[3:6], (((1,), (1,)), ((), ())),
                          preferred_element_type=_f32)
    contrib = jnp.concatenate([c_a, c_b], axis=1)

    @pl.when(i == 0)
    def _():
        s_ref[...] = jnp.zeros_like(s_ref)

    s_ref[...] += contrib

    @pl.when(i == pl.num_programs(0) - 1)
    def _():
        s = s_ref[...]
        ncnt = jnp.maximum(s[:, 5:6], 1.0)
        ecnt = jnp.maximum(s[:, 8:9], 1.0)
        n_mean = s[:, 0:4] / ncnt
        nout_mean = s[:, 4:5] / ncnt
        e_mean = s[:, 6:8] / ecnt
        g1 = jnp.maximum(
            jnp.dot(e_mean, wge[...], preferred_element_type=_f32)
            + jnp.dot(n_mean, wgn[...], preferred_element_type=_f32)
            + blg[...], 0.0)
        z = (jnp.dot(g1, wgg[...], preferred_element_type=_f32)
             + jnp.dot(nout_mean, wgnr[...], preferred_element_type=_f32)
             + brg[...])
        gout_ref[...] = 1.0 / (1.0 + jnp.exp(-z))


def kernel(x, edge_attr, senders, receivers, node_graph,
           We1, be1, We2, be2, Wn1, bn1, Wn2, bn2, bg_enc,
           Wl_e_e, Wl_e_s, Wl_e_g, bl_e,
           Wl_n_n, Wl_n_in, Wl_n_g, bl_n,
           Wl_g_e, Wl_g_n, Wl_g_g, bl_g,
           Wr_n, br_n, Wr_g_g, Wr_g_n, br_g):
    g8 = jnp.maximum(bg_enc, 0.0)
    ble = (bl_e + g8 @ Wl_e_g).reshape(1, 2)
    bln = (bl_n + g8 @ Wl_n_g).reshape(1, 4)
    blg = (bl_g + g8 @ Wl_g_g).reshape(1, 1)

    xp = jnp.pad(x, ((0, NP - N), (0, 0)))
    nt, t0p, t1p = pl.pallas_call(
        _enc_node_body,
        grid=(NP // BN,),
        in_specs=[pl.BlockSpec((BN, 83), lambda i: (i, 0)),
                  _full((83, 64)), _full((1, 64)),
                  _full((64, 32)), _full((32, 1)),
                  _full((32, 2))],
        out_specs=[pl.BlockSpec((32, BN), lambda i: (0, i)),
                   pl.BlockSpec((BN,), lambda i: (i,)),
                   pl.BlockSpec((BN,), lambda i: (i,))],
        out_shape=[jax.ShapeDtypeStruct((32, NP), _f32),
                   jax.ShapeDtypeStruct((NP,), _f32),
                   jax.ShapeDtypeStruct((NP,), _f32)],
    )(xp, Wn1, bn1.reshape(1, 64), Wn2, bn2.reshape(32, 1), Wl_e_s)

    ea0 = jnp.pad(edge_attr[:, 0], (0, EP - E))
    ea1 = jnp.pad(edge_attr[:, 1], (0, EP - E))
    ec0, ec1 = pl.pallas_call(
        _enc_edge_body,
        grid=(EP // BE,),
        in_specs=[pl.BlockSpec((BE,), lambda i: (i,)),
                  pl.BlockSpec((BE,), lambda i: (i,)),
                  _full((2, 4)), _full((1, 4)),
                  _full((4, 16)), _full((1, 16)),
                  _full((16, 2)), _full((1, 2))],
        out_specs=[pl.BlockSpec((BE,), lambda i: (i,)),
                   pl.BlockSpec((BE,), lambda i: (i,))],
        out_shape=[jax.ShapeDtypeStruct((EP,), _f32),
                   jax.ShapeDtypeStruct((EP,), _f32)],
    )(ea0, ea1, We1, be1.reshape(1, 4), We2, be2.reshape(1, 16),
      Wl_e_e, ble)

    spad = jnp.pad(senders.astype(jnp.int32), (0, EP - E),
                   constant_values=N)
    rpad = jnp.pad(receivers.astype(jnp.int32), (0, EP - E),
                   constant_values=N)
    zeros1 = jnp.zeros((NP,), _f32)
    ones1 = jnp.ones((CH,), _f32)
    accf = _sc_edge_phase(spad, rpad, ec0, ec1, t0p, t1p, zeros1, ones1)

    ngp = jnp.pad(node_graph.astype(jnp.int32), (0, NP - N),
                  constant_values=G)
    ng3 = ngp.reshape(NP // BN, 1, BN)
    n_out_p, g_out = pl.pallas_call(
        _finalize_body,
        grid=(NP // BN,),
        in_specs=[pl.BlockSpec((32, BN), lambda i: (0, i)),
                  pl.BlockSpec((12, BN), lambda i: (0, i)),
                  pl.BlockSpec((1, 1, BN), lambda i: (i, 0, 0)),
                  _full((32, 4)), _full((2, 4)), _full((1, 4)),
                  _full((4, 1)), _full((1, 1)),
                  _full((2, 1)), _full((4, 1)), _full((1, 1)),
                  _full((1, 1)), _full((1, 1)), _full((1, 1))],
        out_specs=[pl.BlockSpec((BN, 1), lambda i: (i, 0)),
                   pl.BlockSpec((G, 1), lambda i: (0, 0))],
        out_shape=[jax.ShapeDtypeStruct((NP, 1), _f32),
                   jax.ShapeDtypeStruct((G, 1), _f32)],
        scratch_shapes=[pltpu.VMEM((G, 9), _f32)],
    )(nt, accf.reshape(12, NP), ng3,
      Wl_n_n, Wl_n_in, bln, Wr_n, br_n.reshape(1, 1),
      Wl_g_e, Wl_g_n, blg, Wr_g_g, Wr_g_n, br_g.reshape(1, 1))

    return (n_out_p[:N], g_out)


# speedup vs baseline: 1.2707x; 1.2707x over previous
import functools

import jax
import jax.numpy as jnp
from jax import lax
from jax.experimental import pallas as pl
from jax.experimental.pallas import tpu as pltpu
from jax.experimental.pallas import tpu_sc as plsc

N = 50000
E = 800000
G = 64

NC = 2
NS = 16
NW = NC * NS
NP = 51200
RPT = NP // NS
EP = 819200
CH = 2000
NCHUNKS = E // CH

BN = 1024
BE = 16384

_f32 = jnp.float32


def _enc_node_body(x_ref, w1, b1, w2, b2, ws, nt_ref, t0_ref, t1_ref):
    h = jnp.maximum(jnp.dot(x_ref[...], w1[...],
                            preferred_element_type=_f32) + b1[...], 0.0)
    nnt = jnp.maximum(
        lax.dot_general(w2[...], h, (((0,), (1,)), ((), ())),
                        preferred_element_type=_f32) + b2[...], 0.0)
    nt_ref[...] = nnt
    ns2t = lax.dot_general(ws[...], nnt, (((0,), (0,)), ((), ())),
                           preferred_element_type=_f32)
    t0_ref[...] = ns2t[0]
    t1_ref[...] = ns2t[1]


def _enc_edge_body(a0_ref, a1_ref, w1, b1, w2, b2, we, ble, e0_ref, e1_ref):
    a0 = a0_ref[...]
    a1 = a1_ref[...]
    h1 = [jnp.maximum(a0 * w1[0, j] + a1 * w1[1, j] + b1[0, j], 0.0)
          for j in range(4)]
    h2 = [jnp.maximum(h1[0] * w2[0, k] + h1[1] * w2[1, k]
                      + h1[2] * w2[2, k] + h1[3] * w2[3, k] + b2[0, k], 0.0)
          for k in range(16)]
    for c, ref in ((0, e0_ref), (1, e1_ref)):
        acc = h2[0] * we[0, c]
        for k in range(1, 16):
            acc = acc + h2[k] * we[k, c]
        ref[...] = acc + ble[0, c]


def _full(shape):
    nd = len(shape)
    return pl.BlockSpec(shape, lambda i: (0,) * nd)


def _sc_body(send_hbm, recv_hbm, ec0_hbm, ec1_hbm, t0_hbm, t1_hbm,
             zeros_hbm, ones_hbm, acc_hbm,
             s_v, r_v, e0_v, e1_v, c0_v, c1_v, ones_v, stage_v,
             t0, t1, ai0, ai1, ci, ao0, ao1, co):
    core = lax.axis_index("c")
    sid = lax.axis_index("s")
    wid = sid * NC + core
    r0 = sid * RPT

    pltpu.sync_copy(t0_hbm.at[pl.ds(r0, RPT)], stage_v)
    pltpu.sync_copy(stage_v, t0.at[pl.ds(r0, RPT)])
    pltpu.sync_copy(t1_hbm.at[pl.ds(r0, RPT)], stage_v)
    pltpu.sync_copy(stage_v, t1.at[pl.ds(r0, RPT)])
    pltpu.sync_copy(zeros_hbm.at[pl.ds(r0, RPT)], stage_v)
    for acc in (ai0, ai1, ci, ao0, ao1, co):
        pltpu.sync_copy(stage_v, acc.at[pl.ds(r0, RPT)])
    pltpu.sync_copy(ones_hbm, ones_v)
    plsc.subcore_barrier()

    nloc = (NCHUNKS - wid + NW - 1) // NW

    def chunk(j, carry):
        off = (wid + j * NW) * CH
        pltpu.sync_copy(send_hbm.at[pl.ds(off, CH)], s_v)
        pltpu.sync_copy(recv_hbm.at[pl.ds(off, CH)], r_v)
        pltpu.sync_copy(ec0_hbm.at[pl.ds(off, CH)], e0_v)
        pltpu.sync_copy(ec1_hbm.at[pl.ds(off, CH)], e1_v)
        pltpu.sync_copy(t0.at[s_v], c0_v)
        pltpu.sync_copy(t1.at[s_v], c1_v)

        def vloop(m, c):
            sl = pl.ds(16 * m, 16)
            e0_v[sl] = jnp.maximum(e0_v[sl] + c0_v[sl], 0.0)
            e1_v[sl] = jnp.maximum(e1_v[sl] + c1_v[sl], 0.0)
            return c

        lax.fori_loop(0, CH // 16, vloop, 0, unroll=False)

        pltpu.sync_copy(e0_v, ai0.at[r_v], add=True)
        pltpu.sync_copy(e1_v, ai1.at[r_v], add=True)
        pltpu.sync_copy(ones_v, ci.at[r_v], add=True)
        pltpu.sync_copy(e0_v, ao0.at[s_v], add=True)
        pltpu.sync_copy(e1_v, ao1.at[s_v], add=True)
        pltpu.sync_copy(ones_v, co.at[s_v], add=True)
        return carry

    lax.fori_loop(0, nloc, chunk, 0, unroll=False)
    plsc.subcore_barrier()

    for arr, acc in enumerate((ai0, ai1, ci, ao0, ao1, co)):
        pltpu.sync_copy(acc.at[pl.ds(r0, RPT)], stage_v)
        pltpu.sync_copy(stage_v,
                        acc_hbm.at[pl.ds(core * (6 * NP) + arr * NP + r0,
                                         RPT)])


_sc_edge_phase = functools.partial(
    pl.kernel,
    out_type=jax.ShapeDtypeStruct((NC * 6 * NP,), _f32),
    mesh=plsc.VectorSubcoreMesh(core_axis_name="c", subcore_axis_name="s",
                                num_cores=NC, num_subcores=NS),
    scratch_types=[
        pltpu.VMEM((CH,), jnp.int32),
        pltpu.VMEM((CH,), jnp.int32),
        pltpu.VMEM((CH,), _f32),
        pltpu.VMEM((CH,), _f32),
        pltpu.VMEM((CH,), _f32),
        pltpu.VMEM((CH,), _f32),
        pltpu.VMEM((CH,), _f32),
        pltpu.VMEM((RPT,), _f32),
        pltpu.VMEM_SHARED((NP,), _f32),
        pltpu.VMEM_SHARED((NP,), _f32),
        pltpu.VMEM_SHARED((NP,), _f32),
        pltpu.VMEM_SHARED((NP,), _f32),
        pltpu.VMEM_SHARED((NP,), _f32),
        pltpu.VMEM_SHARED((NP,), _f32),
        pltpu.VMEM_SHARED((NP,), _f32),
        pltpu.VMEM_SHARED((NP,), _f32),
    ],
)(_sc_body)


def _finalize_body(nt_ref, acc_ref, ng_ref,
                   wnn, wnin, bln, wrn, brn, wge, wgn, blg, wgg, wgnr, brg,
                   nout_ref, gout_ref, s_ref):
    i = pl.program_id(0)
    a12 = acc_ref[...]
    a = a12[0:6] + a12[6:12]
    iagg = a[0:2] / jnp.maximum(a[2:3], 1.0)
    n4 = jnp.maximum(
        lax.dot_general(nt_ref[...], wnn[...], (((0,), (0,)), ((), ())),
                        preferred_element_type=_f32)
        + lax.dot_general(iagg, wnin[...], (((0,), (0,)), ((), ())),
                          preferred_element_type=_f32)
        + bln[...], 0.0)
    no = 1.0 / (1.0 + jnp.exp(-(jnp.dot(n4, wrn[...],
                                        preferred_element_type=_f32)
                                + brn[...])))
    nout_ref[...] = no

    ids = ng_ref[0]
    oh = (lax.broadcasted_iota(jnp.int32, (G, BN), 0) == ids).astype(_f32)
    xx = jnp.concatenate([n4, no, jnp.ones((BN, 1), _f32)], axis=1)
    c_a = jnp.dot(oh, xx, preferred_element_type=_f32)
    c_b = lax.dot_general(oh, a[3:6], (((1,), (1,)), ((), ())),
                          preferred_element_type=_f32)
    contrib = jnp.concatenate([c_a, c_b], axis=1)

    @pl.when(i == 0)
    def _():
        s_ref[...] = jnp.zeros_like(s_ref)

    s_ref[...] += contrib

    @pl.when(i == pl.num_programs(0) - 1)
    def _():
        s = s_ref[...]
        ncnt = jnp.maximum(s[:, 5:6], 1.0)
        ecnt = jnp.maximum(s[:, 8:9], 1.0)
        n_mean = s[:, 0:4] / ncnt
        nout_mean = s[:, 4:5] / ncnt
        e_mean = s[:, 6:8] / ecnt
        g1 = jnp.maximum(
            jnp.dot(e_mean, wge[...], preferred_element_type=_f32)
            + jnp.dot(n_mean, wgn[...], preferred_element_type=_f32)
            + blg[...], 0.0)
        z = (jnp.dot(g1, wgg[...], preferred_element_type=_f32)
             + jnp.dot(nout_mean, wgnr[...], preferred_element_type=_f32)
             + brg[...])
        gout_ref[...] = 1.0 / (1.0 + jnp.exp(-z))


def kernel(x, edge_attr, senders, receivers, node_graph,
           We1, be1, We2, be2, Wn1, bn1, Wn2, bn2, bg_enc,
           Wl_e_e, Wl_e_s, Wl_e_g, bl_e,
           Wl_n_n, Wl_n_in, Wl_n_g, bl_n,
           Wl_g_e, Wl_g_n, Wl_g_g, bl_g,
           Wr_n, br_n, Wr_g_g, Wr_g_n, br_g):
    g8 = jnp.maximum(bg_enc, 0.0)
    ble = (bl_e + g8 @ Wl_e_g).reshape(1, 2)
    bln = (bl_n + g8 @ Wl_n_g).reshape(1, 4)
    blg = (bl_g + g8 @ Wl_g_g).reshape(1, 1)

    xp = jnp.pad(x, ((0, NP - N), (0, 0)))
    nt, t0p, t1p = pl.pallas_call(
        _enc_node_body,
        grid=(NP // BN,),
        in_specs=[pl.BlockSpec((BN, 83), lambda i: (i, 0)),
                  _full((83, 64)), _full((1, 64)),
                  _full((64, 32)), _full((32, 1)),
                  _full((32, 2))],
        out_specs=[pl.BlockSpec((32, BN), lambda i: (0, i)),
                   pl.BlockSpec((BN,), lambda i: (i,)),
                   pl.BlockSpec((BN,), lambda i: (i,))],
        out_shape=[jax.ShapeDtypeStruct((32, NP), _f32),
                   jax.ShapeDtypeStruct((NP,), _f32),
                   jax.ShapeDtypeStruct((NP,), _f32)],
    )(xp, Wn1, bn1.reshape(1, 64), Wn2, bn2.reshape(32, 1), Wl_e_s)

    ea0 = jnp.pad(edge_attr[:, 0], (0, EP - E))
    ea1 = jnp.pad(edge_attr[:, 1], (0, EP - E))
    ec0, ec1 = pl.pallas_call(
        _enc_edge_body,
        grid=(EP // BE,),
        in_specs=[pl.BlockSpec((BE,), lambda i: (i,)),
                  pl.BlockSpec((BE,), lambda i: (i,)),
                  _full((2, 4)), _full((1, 4)),
                  _full((4, 16)), _full((1, 16)),
                  _full((16, 2)), _full((1, 2))],
        out_specs=[pl.BlockSpec((BE,), lambda i: (i,)),
                   pl.BlockSpec((BE,), lambda i: (i,))],
        out_shape=[jax.ShapeDtypeStruct((EP,), _f32),
                   jax.ShapeDtypeStruct((EP,), _f32)],
    )(ea0, ea1, We1, be1.reshape(1, 4), We2, be2.reshape(1, 16),
      Wl_e_e, ble)

    zeros1 = jnp.zeros((NP,), _f32)
    ones1 = jnp.ones((CH,), _f32)
    accf = _sc_edge_phase(
        senders.astype(jnp.int32), receivers.astype(jnp.int32),
        ec0, ec1, t0p, t1p, zeros1, ones1)

    ngp = jnp.pad(node_graph.astype(jnp.int32), (0, NP - N),
                  constant_values=G)
    ng3 = ngp.reshape(NP // BN, 1, BN)
    n_out_p, g_out = pl.pallas_call(
        _finalize_body,
        grid=(NP // BN,),
        in_specs=[pl.BlockSpec((32, BN), lambda i: (0, i)),
                  pl.BlockSpec((12, BN), lambda i: (0, i)),
                  pl.BlockSpec((1, 1, BN), lambda i: (i, 0, 0)),
                  _full((32, 4)), _full((2, 4)), _full((1, 4)),
                  _full((4, 1)), _full((1, 1)),
                  _full((2, 1)), _full((4, 1)), _full((1, 1)),
                  _full((1, 1)), _full((1, 1)), _full((1, 1))],
        out_specs=[pl.BlockSpec((BN, 1), lambda i: (i, 0)),
                   pl.BlockSpec((G, 1), lambda i: (0, 0))],
        out_shape=[jax.ShapeDtypeStruct((NP, 1), _f32),
                   jax.ShapeDtypeStruct((G, 1), _f32)],
        scratch_shapes=[pltpu.VMEM((G, 9), _f32)],
    )(nt, accf.reshape(12, NP), ng3,
      Wl_n_n, Wl_n_in, bln, Wr_n, br_n.reshape(1, 1),
      Wl_g_e, Wl_g_n, blg, Wr_g_g, Wr_g_n, br_g.reshape(1, 1))

    return (n_out_p[:N], g_out)


# speedup vs baseline: 1.3977x; 1.0999x over previous
import functools

import jax
import jax.numpy as jnp
from jax import lax
from jax.experimental import pallas as pl
from jax.experimental.pallas import tpu as pltpu
from jax.experimental.pallas import tpu_sc as plsc

N = 50000
E = 800000
G = 64

NC = 2
NS = 16
NW = NC * NS
NP = 51200
RPT = NP // NS
EP = 819200
CH = 2000
NCHUNKS = E // CH

BN = 1024
BF = 2048
BE = 16384

_f32 = jnp.float32


def _enc_node_body(x_ref, w1, b1, w2, b2, ws, nt_ref, t0_ref, t1_ref):
    h = jnp.maximum(jnp.dot(x_ref[...], w1[...],
                            preferred_element_type=_f32) + b1[...], 0.0)
    nnt = jnp.maximum(
        lax.dot_general(w2[...], h, (((0,), (1,)), ((), ())),
                        preferred_element_type=_f32) + b2[...], 0.0)
    nt_ref[...] = nnt
    ns2t = lax.dot_general(ws[...], nnt, (((0,), (0,)), ((), ())),
                           preferred_element_type=_f32)
    t0_ref[...] = ns2t[0]
    t1_ref[...] = ns2t[1]


def _enc_edge_body(a0_ref, a1_ref, w1, b1, w2, b2, we, ble, e0_ref, e1_ref):
    a0 = a0_ref[...]
    a1 = a1_ref[...]
    h1 = [jnp.maximum(a0 * w1[0, j] + a1 * w1[1, j] + b1[0, j], 0.0)
          for j in range(4)]
    h2 = [jnp.maximum(h1[0] * w2[0, k] + h1[1] * w2[1, k]
                      + h1[2] * w2[2, k] + h1[3] * w2[3, k] + b2[0, k], 0.0)
          for k in range(16)]
    for c, ref in ((0, e0_ref), (1, e1_ref)):
        acc = h2[0] * we[0, c]
        for k in range(1, 16):
            acc = acc + h2[k] * we[k, c]
        ref[...] = acc + ble[0, c]


def _full(shape):
    nd = len(shape)
    return pl.BlockSpec(shape, lambda i: (0,) * nd)


def _sc_body(send_hbm, recv_hbm, ec0_hbm, ec1_hbm, t0_hbm, t1_hbm,
             zeros_hbm, ones_hbm, acc_hbm,
             s_v, r_v, e0_v, e1_v, c0_v, c1_v, ones_v, stage_v,
             t0, t1, ai0, ai1, ci, ao0, ao1, co):
    core = lax.axis_index("c")
    sid = lax.axis_index("s")
    wid = sid * NC + core
    r0 = sid * RPT

    pltpu.sync_copy(t0_hbm.at[pl.ds(r0, RPT)], stage_v)
    pltpu.sync_copy(stage_v, t0.at[pl.ds(r0, RPT)])
    pltpu.sync_copy(t1_hbm.at[pl.ds(r0, RPT)], stage_v)
    pltpu.sync_copy(stage_v, t1.at[pl.ds(r0, RPT)])
    pltpu.sync_copy(zeros_hbm.at[pl.ds(r0, RPT)], stage_v)
    for acc in (ai0, ai1, ci, ao0, ao1, co):
        pltpu.sync_copy(stage_v, acc.at[pl.ds(r0, RPT)])
    pltpu.sync_copy(ones_hbm, ones_v)
    plsc.subcore_barrier()

    nloc = (NCHUNKS - wid + NW - 1) // NW

    def chunk(j, carry):
        off = (wid + j * NW) * CH
        pltpu.sync_copy(send_hbm.at[pl.ds(off, CH)], s_v)
        pltpu.sync_copy(recv_hbm.at[pl.ds(off, CH)], r_v)
        pltpu.sync_copy(ec0_hbm.at[pl.ds(off, CH)], e0_v)
        pltpu.sync_copy(ec1_hbm.at[pl.ds(off, CH)], e1_v)
        pltpu.sync_copy(t0.at[s_v], c0_v)
        pltpu.sync_copy(t1.at[s_v], c1_v)

        def vloop(m, c):
            sl = pl.ds(16 * m, 16)
            e0_v[sl] = jnp.maximum(e0_v[sl] + c0_v[sl], 0.0)
            e1_v[sl] = jnp.maximum(e1_v[sl] + c1_v[sl], 0.0)
            return c

        lax.fori_loop(0, CH // 16, vloop, 0, unroll=False)

        pltpu.sync_copy(e0_v, ai0.at[r_v], add=True)
        pltpu.sync_copy(e1_v, ai1.at[r_v], add=True)
        pltpu.sync_copy(ones_v, ci.at[r_v], add=True)
        pltpu.sync_copy(e0_v, ao0.at[s_v], add=True)
        pltpu.sync_copy(e1_v, ao1.at[s_v], add=True)
        pltpu.sync_copy(ones_v, co.at[s_v], add=True)
        return carry

    lax.fori_loop(0, nloc, chunk, 0, unroll=False)
    plsc.subcore_barrier()

    for arr, acc in enumerate((ai0, ai1, ci, ao0, ao1, co)):
        pltpu.sync_copy(acc.at[pl.ds(r0, RPT)], stage_v)
        pltpu.sync_copy(stage_v,
                        acc_hbm.at[pl.ds(core * (6 * NP) + arr * NP + r0,
                                         RPT)])


_sc_edge_phase = functools.partial(
    pl.kernel,
    out_type=jax.ShapeDtypeStruct((NC * 6 * NP,), _f32),
    mesh=plsc.VectorSubcoreMesh(core_axis_name="c", subcore_axis_name="s",
                                num_cores=NC, num_subcores=NS),
    scratch_types=[
        pltpu.VMEM((CH,), jnp.int32),
        pltpu.VMEM((CH,), jnp.int32),
        pltpu.VMEM((CH,), _f32),
        pltpu.VMEM((CH,), _f32),
        pltpu.VMEM((CH,), _f32),
        pltpu.VMEM((CH,), _f32),
        pltpu.VMEM((CH,), _f32),
        pltpu.VMEM((RPT,), _f32),
        pltpu.VMEM_SHARED((NP,), _f32),
        pltpu.VMEM_SHARED((NP,), _f32),
        pltpu.VMEM_SHARED((NP,), _f32),
        pltpu.VMEM_SHARED((NP,), _f32),
        pltpu.VMEM_SHARED((NP,), _f32),
        pltpu.VMEM_SHARED((NP,), _f32),
        pltpu.VMEM_SHARED((NP,), _f32),
        pltpu.VMEM_SHARED((NP,), _f32),
    ],
)(_sc_body)


def _finalize_body(nt_ref, acc_ref, ng_ref,
                   wnn, wnin, bln, wrn, brn, wge, wgn, blg, wgg, wgnr, brg,
                   nout_ref, gout_ref, s_ref):
    i = pl.program_id(0)
    a12 = acc_ref[...]
    a = a12[0:6] + a12[6:12]
    iagg = a[0:2] / jnp.maximum(a[2:3], 1.0)
    n4 = jnp.maximum(
        lax.dot_general(nt_ref[...], wnn[...], (((0,), (0,)), ((), ())),
                        preferred_element_type=_f32)
        + lax.dot_general(iagg, wnin[...], (((0,), (0,)), ((), ())),
                          preferred_element_type=_f32)
        + bln[...], 0.0)
    no = 1.0 / (1.0 + jnp.exp(-(jnp.dot(n4, wrn[...],
                                        preferred_element_type=_f32)
                                + brn[...])))
    nout_ref[...] = no

    ids = ng_ref[0]
    oh = (lax.broadcasted_iota(jnp.int32, (G, BF), 0) == ids).astype(_f32)
    xx = jnp.concatenate([n4, no, jnp.ones((BF, 1), _f32)], axis=1)
    c_a = jnp.dot(oh, xx, preferred_element_type=_f32)
    c_b = lax.dot_general(oh, a[3:6], (((1,), (1,)), ((), ())),
                          preferred_element_type=_f32)
    contrib = jnp.concatenate([c_a, c_b], axis=1)

    @pl.when(i == 0)
    def _():
        s_ref[...] = jnp.zeros_like(s_ref)

    s_ref[...] += contrib

    @pl.when(i == pl.num_programs(0) - 1)
    def _():
        s = s_ref[...]
        ncnt = jnp.maximum(s[:, 5:6], 1.0)
        ecnt = jnp.maximum(s[:, 8:9], 1.0)
        n_mean = s[:, 0:4] / ncnt
        nout_mean = s[:, 4:5] / ncnt
        e_mean = s[:, 6:8] / ecnt
        g1 = jnp.maximum(
            jnp.dot(e_mean, wge[...], preferred_element_type=_f32)
            + jnp.dot(n_mean, wgn[...], preferred_element_type=_f32)
            + blg[...], 0.0)
        z = (jnp.dot(g1, wgg[...], preferred_element_type=_f32)
             + jnp.dot(nout_mean, wgnr[...], preferred_element_type=_f32)
             + brg[...])
        gout_ref[...] = 1.0 / (1.0 + jnp.exp(-z))


def kernel(x, edge_attr, senders, receivers, node_graph,
           We1, be1, We2, be2, Wn1, bn1, Wn2, bn2, bg_enc,
           Wl_e_e, Wl_e_s, Wl_e_g, bl_e,
           Wl_n_n, Wl_n_in, Wl_n_g, bl_n,
           Wl_g_e, Wl_g_n, Wl_g_g, bl_g,
           Wr_n, br_n, Wr_g_g, Wr_g_n, br_g):
    g8 = jnp.maximum(bg_enc, 0.0)
    ble = (bl_e + g8 @ Wl_e_g).reshape(1, 2)
    bln = (bl_n + g8 @ Wl_n_g).reshape(1, 4)
    blg = (bl_g + g8 @ Wl_g_g).reshape(1, 1)

    xp = jnp.pad(x, ((0, NP - N), (0, 0)))
    nt, t0p, t1p = pl.pallas_call(
        _enc_node_body,
        grid=(NP // BN,),
        in_specs=[pl.BlockSpec((BN, 83), lambda i: (i, 0)),
                  _full((83, 64)), _full((1, 64)),
                  _full((64, 32)), _full((32, 1)),
                  _full((32, 2))],
        out_specs=[pl.BlockSpec((32, BN), lambda i: (0, i)),
                   pl.BlockSpec((BN,), lambda i: (i,)),
                   pl.BlockSpec((BN,), lambda i: (i,))],
        out_shape=[jax.ShapeDtypeStruct((32, NP), _f32),
                   jax.ShapeDtypeStruct((NP,), _f32),
                   jax.ShapeDtypeStruct((NP,), _f32)],
    )(xp, Wn1, bn1.reshape(1, 64), Wn2, bn2.reshape(32, 1), Wl_e_s)

    ea0 = jnp.pad(edge_attr[:, 0], (0, EP - E))
    ea1 = jnp.pad(edge_attr[:, 1], (0, EP - E))
    ec0, ec1 = pl.pallas_call(
        _enc_edge_body,
        grid=(EP // BE,),
        in_specs=[pl.BlockSpec((BE,), lambda i: (i,)),
                  pl.BlockSpec((BE,), lambda i: (i,)),
                  _full((2, 4)), _full((1, 4)),
                  _full((4, 16)), _full((1, 16)),
                  _full((16, 2)), _full((1, 2))],
        out_specs=[pl.BlockSpec((BE,), lambda i: (i,)),
                   pl.BlockSpec((BE,), lambda i: (i,))],
        out_shape=[jax.ShapeDtypeStruct((EP,), _f32),
                   jax.ShapeDtypeStruct((EP,), _f32)],
    )(ea0, ea1, We1, be1.reshape(1, 4), We2, be2.reshape(1, 16),
      Wl_e_e, ble)

    zeros1 = jnp.zeros((NP,), _f32)
    ones1 = jnp.ones((CH,), _f32)
    accf = _sc_edge_phase(
        senders.astype(jnp.int32), receivers.astype(jnp.int32),
        ec0, ec1, t0p, t1p, zeros1, ones1)

    ngp = jnp.pad(node_graph.astype(jnp.int32), (0, NP - N),
                  constant_values=G)
    ng3 = ngp.reshape(NP // BF, 1, BF)
    n_out, g_out = pl.pallas_call(
        _finalize_body,
        grid=(NP // BF,),
        in_specs=[pl.BlockSpec((32, BF), lambda i: (0, i)),
                  pl.BlockSpec((12, BF), lambda i: (0, i)),
                  pl.BlockSpec((1, 1, BF), lambda i: (i, 0, 0)),
                  _full((32, 4)), _full((2, 4)), _full((1, 4)),
                  _full((4, 1)), _full((1, 1)),
                  _full((2, 1)), _full((4, 1)), _full((1, 1)),
                  _full((1, 1)), _full((1, 1)), _full((1, 1))],
        out_specs=[pl.BlockSpec((BF, 1), lambda i: (i, 0)),
                   pl.BlockSpec((G, 1), lambda i: (0, 0))],
        out_shape=[jax.ShapeDtypeStruct((N, 1), _f32),
                   jax.ShapeDtypeStruct((G, 1), _f32)],
        scratch_shapes=[pltpu.VMEM((G, 9), _f32)],
    )(nt, accf.reshape(12, NP), ng3,
      Wl_n_n, Wl_n_in, bln, Wr_n, br_n.reshape(1, 1),
      Wl_g_e, Wl_g_n, blg, Wr_g_g, Wr_g_n, br_g.reshape(1, 1))

    return (n_out, g_out)


# speedup vs baseline: 1.5149x; 1.0839x over previous
import functools

import jax
import jax.numpy as jnp
from jax import lax
from jax.experimental import pallas as pl
from jax.experimental.pallas import tpu as pltpu
from jax.experimental.pallas import tpu_sc as plsc

N = 50000
E = 800000
G = 64

NC = 2
NS = 16
NW = NC * NS
NP = 51200
RPT = NP // NS
EP = 819200
CH = 2000
NCHUNKS = E // CH

BN = 1024
BF = 2048
BE = 16384

_f32 = jnp.float32


def _enc_node_body(x_ref, w1, b1, w2, b2, ws, nt_ref, t0_ref, t1_ref):
    h = jnp.maximum(jnp.dot(x_ref[...], w1[...],
                            preferred_element_type=_f32) + b1[...], 0.0)
    nnt = jnp.maximum(
        lax.dot_general(w2[...], h, (((0,), (1,)), ((), ())),
                        preferred_element_type=_f32) + b2[...], 0.0)
    nt_ref[...] = nnt
    ns2t = lax.dot_general(ws[...], nnt, (((0,), (0,)), ((), ())),
                           preferred_element_type=_f32)
    t0_ref[...] = ns2t[0]
    t1_ref[...] = ns2t[1]


def _enc_edge_body(a0_ref, a1_ref, w1, b1, w2, b2, we, ble, e0_ref, e1_ref):
    a0 = a0_ref[...]
    a1 = a1_ref[...]
    h1 = [jnp.maximum(a0 * w1[0, j] + a1 * w1[1, j] + b1[0, j], 0.0)
          for j in range(4)]
    h2 = [jnp.maximum(h1[0] * w2[0, k] + h1[1] * w2[1, k]
                      + h1[2] * w2[2, k] + h1[3] * w2[3, k] + b2[0, k], 0.0)
          for k in range(16)]
    for c, ref in ((0, e0_ref), (1, e1_ref)):
        acc = h2[0] * we[0, c]
        for k in range(1, 16):
            acc = acc + h2[k] * we[k, c]
        ref[...] = acc + ble[0, c]


def _full(shape):
    nd = len(shape)
    return pl.BlockSpec(shape, lambda i: (0,) * nd)


def _sc_body(send_hbm, recv_hbm, ec0_hbm, ec1_hbm, t0_hbm, t1_hbm,
             zeros_hbm, ones_hbm, acc_hbm,
             s_v, r_v, e0_v, e1_v, c0_v, c1_v, ones_v, stage_v,
             sem_l, sem_g, sem_s,
             t0, t1, ai0, ai1, ci, ao0, ao1, co):
    core = lax.axis_index("c")
    sid = lax.axis_index("s")
    wid = sid * NC + core
    r0 = sid * RPT

    pltpu.sync_copy(t0_hbm.at[pl.ds(r0, RPT)], stage_v)
    pltpu.sync_copy(stage_v, t0.at[pl.ds(r0, RPT)])
    pltpu.sync_copy(t1_hbm.at[pl.ds(r0, RPT)], stage_v)
    pltpu.sync_copy(stage_v, t1.at[pl.ds(r0, RPT)])
    pltpu.sync_copy(zeros_hbm.at[pl.ds(r0, RPT)], stage_v)
    for acc in (ai0, ai1, ci, ao0, ao1, co):
        pltpu.sync_copy(stage_v, acc.at[pl.ds(r0, RPT)])
    pltpu.sync_copy(ones_hbm, ones_v)
    plsc.subcore_barrier()

    nloc = (NCHUNKS - wid + NW - 1) // NW

    def chunk(j, carry):
        off = (wid + j * NW) * CH
        ld_s = pltpu.async_copy(send_hbm.at[pl.ds(off, CH)], s_v, sem_l)
        ld_r = pltpu.async_copy(recv_hbm.at[pl.ds(off, CH)], r_v, sem_l)
        ld_0 = pltpu.async_copy(ec0_hbm.at[pl.ds(off, CH)], e0_v, sem_l)
        ld_1 = pltpu.async_copy(ec1_hbm.at[pl.ds(off, CH)], e1_v, sem_l)
        ld_s.wait()
        g_0 = pltpu.async_copy(t0.at[s_v], c0_v, sem_g)
        g_1 = pltpu.async_copy(t1.at[s_v], c1_v, sem_g)
        ld_r.wait()
        ld_0.wait()
        ld_1.wait()
        g_0.wait()
        g_1.wait()

        def vloop(m, c):
            sl = pl.ds(16 * m, 16)
            e0_v[sl] = jnp.maximum(e0_v[sl] + c0_v[sl], 0.0)
            e1_v[sl] = jnp.maximum(e1_v[sl] + c1_v[sl], 0.0)
            return c

        lax.fori_loop(0, CH // 16, vloop, 0, unroll=False)

        sc = [pltpu.async_copy(e0_v, ai0.at[r_v], sem_s, add=True),
              pltpu.async_copy(e1_v, ai1.at[r_v], sem_s, add=True),
              pltpu.async_copy(ones_v, ci.at[r_v], sem_s, add=True),
              pltpu.async_copy(e0_v, ao0.at[s_v], sem_s, add=True),
              pltpu.async_copy(e1_v, ao1.at[s_v], sem_s, add=True),
              pltpu.async_copy(ones_v, co.at[s_v], sem_s, add=True)]
        for d in sc:
            d.wait()
        return carry

    lax.fori_loop(0, nloc, chunk, 0, unroll=False)
    plsc.subcore_barrier()

    for arr, acc in enumerate((ai0, ai1, ci, ao0, ao1, co)):
        pltpu.sync_copy(acc.at[pl.ds(r0, RPT)], stage_v)
        pltpu.sync_copy(stage_v,
                        acc_hbm.at[pl.ds(core * (6 * NP) + arr * NP + r0,
                                         RPT)])


_sc_edge_phase = functools.partial(
    pl.kernel,
    out_type=jax.ShapeDtypeStruct((NC * 6 * NP,), _f32),
    mesh=plsc.VectorSubcoreMesh(core_axis_name="c", subcore_axis_name="s",
                                num_cores=NC, num_subcores=NS),
    scratch_types=[
        pltpu.VMEM((CH,), jnp.int32),
        pltpu.VMEM((CH,), jnp.int32),
        pltpu.VMEM((CH,), _f32),
        pltpu.VMEM((CH,), _f32),
        pltpu.VMEM((CH,), _f32),
        pltpu.VMEM((CH,), _f32),
        pltpu.VMEM((CH,), _f32),
        pltpu.VMEM((RPT,), _f32),
        pltpu.SemaphoreType.DMA,
        pltpu.SemaphoreType.DMA,
        pltpu.SemaphoreType.DMA,
        pltpu.VMEM_SHARED((NP,), _f32),
        pltpu.VMEM_SHARED((NP,), _f32),
        pltpu.VMEM_SHARED((NP,), _f32),
        pltpu.VMEM_SHARED((NP,), _f32),
        pltpu.VMEM_SHARED((NP,), _f32),
        pltpu.VMEM_SHARED((NP,), _f32),
        pltpu.VMEM_SHARED((NP,), _f32),
        pltpu.VMEM_SHARED((NP,), _f32),
    ],
)(_sc_body)


def _finalize_body(nt_ref, acc_ref, ng_ref,
                   wnn, wnin, bln, wrn, brn, wge, wgn, blg, wgg, wgnr, brg,
                   nout_ref, gout_ref, s_ref):
    i = pl.program_id(0)
    a12 = acc_ref[...]
    a = a12[0:6] + a12[6:12]
    iagg = a[0:2] / jnp.maximum(a[2:3], 1.0)
    n4 = jnp.maximum(
        lax.dot_general(nt_ref[...], wnn[...], (((0,), (0,)), ((), ())),
                        preferred_element_type=_f32)
        + lax.dot_general(iagg, wnin[...], (((0,), (0,)), ((), ())),
                          preferred_element_type=_f32)
        + bln[...], 0.0)
    no = 1.0 / (1.0 + jnp.exp(-(jnp.dot(n4, wrn[...],
                                        preferred_element_type=_f32)
                                + brn[...])))
    nout_ref[...] = no

    ids = ng_ref[0]
    oh = (lax.broadcasted_iota(jnp.int32, (G, BF), 0) == ids).astype(_f32)
    xx = jnp.concatenate([n4, no, jnp.ones((BF, 1), _f32)], axis=1)
    c_a = jnp.dot(oh, xx, preferred_element_type=_f32)
    c_b = lax.dot_general(oh, a[3:6], (((1,), (1,)), ((), ())),
                          preferred_element_type=_f32)
    contrib = jnp.concatenate([c_a, c_b], axis=1)

    @pl.when(i == 0)
    def _():
        s_ref[...] = jnp.zeros_like(s_ref)

    s_ref[...] += contrib

    @pl.when(i == pl.num_programs(0) - 1)
    def _():
        s = s_ref[...]
        ncnt = jnp.maximum(s[:, 5:6], 1.0)
        ecnt = jnp.maximum(s[:, 8:9], 1.0)
        n_mean = s[:, 0:4] / ncnt
        nout_mean = s[:, 4:5] / ncnt
        e_mean = s[:, 6:8] / ecnt
        g1 = jnp.maximum(
            jnp.dot(e_mean, wge[...], preferred_element_type=_f32)
            + jnp.dot(n_mean, wgn[...], preferred_element_type=_f32)
            + blg[...], 0.0)
        z = (jnp.dot(g1, wgg[...], preferred_element_type=_f32)
             + jnp.dot(nout_mean, wgnr[...], preferred_element_type=_f32)
             + brg[...])
        gout_ref[...] = 1.0 / (1.0 + jnp.exp(-z))


def kernel(x, edge_attr, senders, receivers, node_graph,
           We1, be1, We2, be2, Wn1, bn1, Wn2, bn2, bg_enc,
           Wl_e_e, Wl_e_s, Wl_e_g, bl_e,
           Wl_n_n, Wl_n_in, Wl_n_g, bl_n,
           Wl_g_e, Wl_g_n, Wl_g_g, bl_g,
           Wr_n, br_n, Wr_g_g, Wr_g_n, br_g):
    g8 = jnp.maximum(bg_enc, 0.0)
    ble = (bl_e + g8 @ Wl_e_g).reshape(1, 2)
    bln = (bl_n + g8 @ Wl_n_g).reshape(1, 4)
    blg = (bl_g + g8 @ Wl_g_g).reshape(1, 1)

    xp = jnp.pad(x, ((0, NP - N), (0, 0)))
    nt, t0p, t1p = pl.pallas_call(
        _enc_node_body,
        grid=(NP // BN,),
        in_specs=[pl.BlockSpec((BN, 83), lambda i: (i, 0)),
                  _full((83, 64)), _full((1, 64)),
                  _full((64, 32)), _full((32, 1)),
                  _full((32, 2))],
        out_specs=[pl.BlockSpec((32, BN), lambda i: (0, i)),
                   pl.BlockSpec((BN,), lambda i: (i,)),
                   pl.BlockSpec((BN,), lambda i: (i,))],
        out_shape=[jax.ShapeDtypeStruct((32, NP), _f32),
                   jax.ShapeDtypeStruct((NP,), _f32),
                   jax.ShapeDtypeStruct((NP,), _f32)],
    )(xp, Wn1, bn1.reshape(1, 64), Wn2, bn2.reshape(32, 1), Wl_e_s)

    ea0 = jnp.pad(edge_attr[:, 0], (0, EP - E))
    ea1 = jnp.pad(edge_attr[:, 1], (0, EP - E))
    ec0, ec1 = pl.pallas_call(
        _enc_edge_body,
        grid=(EP // BE,),
        in_specs=[pl.BlockSpec((BE,), lambda i: (i,)),
                  pl.BlockSpec((BE,), lambda i: (i,)),
                  _full((2, 4)), _full((1, 4)),
                  _full((4, 16)), _full((1, 16)),
                  _full((16, 2)), _full((1, 2))],
        out_specs=[pl.BlockSpec((BE,), lambda i: (i,)),
                   pl.BlockSpec((BE,), lambda i: (i,))],
        out_shape=[jax.ShapeDtypeStruct((EP,), _f32),
                   jax.ShapeDtypeStruct((EP,), _f32)],
    )(ea0, ea1, We1, be1.reshape(1, 4), We2, be2.reshape(1, 16),
      Wl_e_e, ble)

    zeros1 = jnp.zeros((NP,), _f32)
    ones1 = jnp.ones((CH,), _f32)
    accf = _sc_edge_phase(
        senders.astype(jnp.int32), receivers.astype(jnp.int32),
        ec0, ec1, t0p, t1p, zeros1, ones1)

    ngp = jnp.pad(node_graph.astype(jnp.int32), (0, NP - N),
                  constant_values=G)
    ng3 = ngp.reshape(NP // BF, 1, BF)
    n_out, g_out = pl.pallas_call(
        _finalize_body,
        grid=(NP // BF,),
        in_specs=[pl.BlockSpec((32, BF), lambda i: (0, i)),
                  pl.BlockSpec((12, BF), lambda i: (0, i)),
                  pl.BlockSpec((1, 1, BF), lambda i: (i, 0, 0)),
                  _full((32, 4)), _full((2, 4)), _full((1, 4)),
                  _full((4, 1)), _full((1, 1)),
                  _full((2, 1)), _full((4, 1)), _full((1, 1)),
                  _full((1, 1)), _full((1, 1)), _full((1, 1))],
        out_specs=[pl.BlockSpec((BF, 1), lambda i: (i, 0)),
                   pl.BlockSpec((G, 1), lambda i: (0, 0))],
        out_shape=[jax.ShapeDtypeStruct((N, 1), _f32),
                   jax.ShapeDtypeStruct((G, 1), _f32)],
        scratch_shapes=[pltpu.VMEM((G, 9), _f32)],
    )(nt, accf.reshape(12, NP), ng3,
      Wl_n_n, Wl_n_in, bln, Wr_n, br_n.reshape(1, 1),
      Wl_g_e, Wl_g_n, blg, Wr_g_g, Wr_g_n, br_g.reshape(1, 1))

    return (n_out, g_out)


# speedup vs baseline: 1.6106x; 1.0632x over previous
import functools

import jax
import jax.numpy as jnp
from jax import lax
from jax.experimental import pallas as pl
from jax.experimental.pallas import tpu as pltpu
from jax.experimental.pallas import tpu_sc as plsc

N = 50000
E = 800000
G = 64

NC = 2
NS = 16
NW = NC * NS
NP = 51200
RPT = NP // NS
EP = 819200
CH = 2000
NCHUNKS = E // CH

BN = 1024
BF = 2048
BE = 16384

_f32 = jnp.float32


def _enc_node_body(xt_ref, w1, b1, w2, b2, ws, nt_ref, t0_ref, t1_ref):
    h = jnp.maximum(
        lax.dot_general(w1[...], xt_ref[...], (((0,), (0,)), ((), ())),
                        preferred_element_type=_f32) + b1[...], 0.0)
    nnt = jnp.maximum(
        lax.dot_general(w2[...], h, (((0,), (0,)), ((), ())),
                        preferred_element_type=_f32) + b2[...], 0.0)
    nt_ref[...] = nnt
    ns2t = lax.dot_general(ws[...], nnt, (((0,), (0,)), ((), ())),
                           preferred_element_type=_f32)
    t0_ref[...] = ns2t[0]
    t1_ref[...] = ns2t[1]


def _enc_edge_body(a0_ref, a1_ref, w1, b1, w2, b2, we, ble, e0_ref, e1_ref):
    a0 = a0_ref[...]
    a1 = a1_ref[...]
    h1 = [jnp.maximum(a0 * w1[0, j] + a1 * w1[1, j] + b1[0, j], 0.0)
          for j in range(4)]
    h2 = [jnp.maximum(h1[0] * w2[0, k] + h1[1] * w2[1, k]
                      + h1[2] * w2[2, k] + h1[3] * w2[3, k] + b2[0, k], 0.0)
          for k in range(16)]
    for c, ref in ((0, e0_ref), (1, e1_ref)):
        acc = h2[0] * we[0, c]
        for k in range(1, 16):
            acc = acc + h2[k] * we[k, c]
        ref[...] = acc + ble[0, c]


def _full(shape):
    nd = len(shape)
    return pl.BlockSpec(shape, lambda i: (0,) * nd)


def _sc_body(send_hbm, recv_hbm, ec0_hbm, ec1_hbm, t0_hbm, t1_hbm,
             zeros_hbm, ones_hbm, acc_hbm,
             s_v, r_v, e0_v, e1_v, c0_v, c1_v, ones_v, stage_v,
             sem_l, sem_g, sem_s,
             t0, t1, ai0, ai1, ci, ao0, ao1, co):
    core = lax.axis_index("c")
    sid = lax.axis_index("s")
    wid = sid * NC + core
    r0 = sid * RPT

    pltpu.sync_copy(t0_hbm.at[pl.ds(r0, RPT)], stage_v)
    pltpu.sync_copy(stage_v, t0.at[pl.ds(r0, RPT)])
    pltpu.sync_copy(t1_hbm.at[pl.ds(r0, RPT)], stage_v)
    pltpu.sync_copy(stage_v, t1.at[pl.ds(r0, RPT)])
    pltpu.sync_copy(zeros_hbm.at[pl.ds(r0, RPT)], stage_v)
    for acc in (ai0, ai1, ci, ao0, ao1, co):
        pltpu.sync_copy(stage_v, acc.at[pl.ds(r0, RPT)])
    pltpu.sync_copy(ones_hbm, ones_v)
    plsc.subcore_barrier()

    nloc = (NCHUNKS - wid + NW - 1) // NW

    def chunk(j, carry):
        off = (wid + j * NW) * CH
        ld_s = pltpu.async_copy(send_hbm.at[pl.ds(off, CH)], s_v, sem_l)
        ld_r = pltpu.async_copy(recv_hbm.at[pl.ds(off, CH)], r_v, sem_l)
        ld_0 = pltpu.async_copy(ec0_hbm.at[pl.ds(off, CH)], e0_v, sem_l)
        ld_1 = pltpu.async_copy(ec1_hbm.at[pl.ds(off, CH)], e1_v, sem_l)
        ld_s.wait()
        g_0 = pltpu.async_copy(t0.at[s_v], c0_v, sem_g)
        g_1 = pltpu.async_copy(t1.at[s_v], c1_v, sem_g)
        ld_r.wait()
        ld_0.wait()
        ld_1.wait()
        g_0.wait()
        g_1.wait()

        def vloop(m, c):
            sl = pl.ds(16 * m, 16)
            e0_v[sl] = jnp.maximum(e0_v[sl] + c0_v[sl], 0.0)
            e1_v[sl] = jnp.maximum(e1_v[sl] + c1_v[sl], 0.0)
            return c

        lax.fori_loop(0, CH // 16, vloop, 0, unroll=False)

        sc = [pltpu.async_copy(e0_v, ai0.at[r_v], sem_s, add=True),
              pltpu.async_copy(e1_v, ai1.at[r_v], sem_s, add=True),
              pltpu.async_copy(ones_v, ci.at[r_v], sem_s, add=True),
              pltpu.async_copy(e0_v, ao0.at[s_v], sem_s, add=True),
              pltpu.async_copy(e1_v, ao1.at[s_v], sem_s, add=True),
              pltpu.async_copy(ones_v, co.at[s_v], sem_s, add=True)]
        for d in sc:
            d.wait()
        return carry

    lax.fori_loop(0, nloc, chunk, 0, unroll=False)
    plsc.subcore_barrier()

    for arr, acc in enumerate((ai0, ai1, ci, ao0, ao1, co)):
        pltpu.sync_copy(acc.at[pl.ds(r0, RPT)], stage_v)
        pltpu.sync_copy(stage_v,
                        acc_hbm.at[pl.ds(core * (6 * NP) + arr * NP + r0,
                                         RPT)])


_sc_edge_phase = functools.partial(
    pl.kernel,
    out_type=jax.ShapeDtypeStruct((NC * 6 * NP,), _f32),
    mesh=plsc.VectorSubcoreMesh(core_axis_name="c", subcore_axis_name="s",
                                num_cores=NC, num_subcores=NS),
    scratch_types=[
        pltpu.VMEM((CH,), jnp.int32),
        pltpu.VMEM((CH,), jnp.int32),
        pltpu.VMEM((CH,), _f32),
        pltpu.VMEM((CH,), _f32),
        pltpu.VMEM((CH,), _f32),
        pltpu.VMEM((CH,), _f32),
        pltpu.VMEM((CH,), _f32),
        pltpu.VMEM((RPT,), _f32),
        pltpu.SemaphoreType.DMA,
        pltpu.SemaphoreType.DMA,
        pltpu.SemaphoreType.DMA,
        pltpu.VMEM_SHARED((NP,), _f32),
        pltpu.VMEM_SHARED((NP,), _f32),
        pltpu.VMEM_SHARED((NP,), _f32),
        pltpu.VMEM_SHARED((NP,), _f32),
        pltpu.VMEM_SHARED((NP,), _f32),
        pltpu.VMEM_SHARED((NP,), _f32),
        pltpu.VMEM_SHARED((NP,), _f32),
        pltpu.VMEM_SHARED((NP,), _f32),
    ],
)(_sc_body)


def _finalize_body(nt_ref, acc_ref, ng_ref,
                   wnn, wnin, bln, wrn, brn, wge, wgn, blg, wgg, wgnr, brg,
                   nout_ref, gout_ref, s_ref):
    i = pl.program_id(0)
    a12 = acc_ref[...]
    a = a12[0:6] + a12[6:12]
    iagg = a[0:2] / jnp.maximum(a[2:3], 1.0)
    n4 = jnp.maximum(
        lax.dot_general(nt_ref[...], wnn[...], (((0,), (0,)), ((), ())),
                        preferred_element_type=_f32)
        + lax.dot_general(iagg, wnin[...], (((0,), (0,)), ((), ())),
                          preferred_element_type=_f32)
        + bln[...], 0.0)
    n4 = jnp.minimum(n4, 1e30)
    n4 = jnp.where(n4 == n4, n4, 0.0)
    no = 1.0 / (1.0 + jnp.exp(-(jnp.dot(n4, wrn[...],
                                        preferred_element_type=_f32)
                                + brn[...])))
    nout_ref[...] = no

    ids = ng_ref[0]
    oh = (lax.broadcasted_iota(jnp.int32, (G, BF), 0) == ids).astype(_f32)
    xx = jnp.concatenate([n4, no, jnp.ones((BF, 1), _f32)], axis=1)
    c_a = jnp.dot(oh, xx, preferred_element_type=_f32)
    c_b = lax.dot_general(oh, a[3:6], (((1,), (1,)), ((), ())),
                          preferred_element_type=_f32)
    contrib = jnp.concatenate([c_a, c_b], axis=1)

    @pl.when(i == 0)
    def _():
        s_ref[...] = jnp.zeros_like(s_ref)

    s_ref[...] += contrib

    @pl.when(i == pl.num_programs(0) - 1)
    def _():
        s = s_ref[...]
        ncnt = jnp.maximum(s[:, 5:6], 1.0)
        ecnt = jnp.maximum(s[:, 8:9], 1.0)
        n_mean = s[:, 0:4] / ncnt
        nout_mean = s[:, 4:5] / ncnt
        e_mean = s[:, 6:8] / ecnt
        g1 = jnp.maximum(
            jnp.dot(e_mean, wge[...], preferred_element_type=_f32)
            + jnp.dot(n_mean, wgn[...], preferred_element_type=_f32)
            + blg[...], 0.0)
        z = (jnp.dot(g1, wgg[...], preferred_element_type=_f32)
             + jnp.dot(nout_mean, wgnr[...], preferred_element_type=_f32)
             + brg[...])
        gout_ref[...] = 1.0 / (1.0 + jnp.exp(-z))


def kernel(x, edge_attr, senders, receivers, node_graph,
           We1, be1, We2, be2, Wn1, bn1, Wn2, bn2, bg_enc,
           Wl_e_e, Wl_e_s, Wl_e_g, bl_e,
           Wl_n_n, Wl_n_in, Wl_n_g, bl_n,
           Wl_g_e, Wl_g_n, Wl_g_g, bl_g,
           Wr_n, br_n, Wr_g_g, Wr_g_n, br_g):
    g8 = jnp.maximum(bg_enc, 0.0)
    ble = (bl_e + g8 @ Wl_e_g).reshape(1, 2)
    bln = (bl_n + g8 @ Wl_n_g).reshape(1, 4)
    blg = (bl_g + g8 @ Wl_g_g).reshape(1, 1)

    nt, t0p, t1p = pl.pallas_call(
        _enc_node_body,
        grid=(49,),
        in_specs=[pl.BlockSpec((83, BN), lambda i: (0, i)),
                  _full((83, 64)), _full((64, 1)),
                  _full((64, 32)), _full((32, 1)),
                  _full((32, 2))],
        out_specs=[pl.BlockSpec((32, BN), lambda i: (0, i)),
                   pl.BlockSpec((BN,), lambda i: (i,)),
                   pl.BlockSpec((BN,), lambda i: (i,))],
        out_shape=[jax.ShapeDtypeStruct((32, NP), _f32),
                   jax.ShapeDtypeStruct((NP,), _f32),
                   jax.ShapeDtypeStruct((NP,), _f32)],
    )(x.T, Wn1, bn1.reshape(64, 1), Wn2, bn2.reshape(32, 1), Wl_e_s)

    ea0 = jnp.pad(edge_attr[:, 0], (0, EP - E))
    ea1 = jnp.pad(edge_attr[:, 1], (0, EP - E))
    ec0, ec1 = pl.pallas_call(
        _enc_edge_body,
        grid=(EP // BE,),
        in_specs=[pl.BlockSpec((BE,), lambda i: (i,)),
                  pl.BlockSpec((BE,), lambda i: (i,)),
                  _full((2, 4)), _full((1, 4)),
                  _full((4, 16)), _full((1, 16)),
                  _full((16, 2)), _full((1, 2))],
        out_specs=[pl.BlockSpec((BE,), lambda i: (i,)),
                   pl.BlockSpec((BE,), lambda i: (i,))],
        out_shape=[jax.ShapeDtypeStruct((EP,), _f32),
                   jax.ShapeDtypeStruct((EP,), _f32)],
    )(ea0, ea1, We1, be1.reshape(1, 4), We2, be2.reshape(1, 16),
      Wl_e_e, ble)

    zeros1 = jnp.zeros((NP,), _f32)
    ones1 = jnp.ones((CH,), _f32)
    accf = _sc_edge_phase(
        senders.astype(jnp.int32), receivers.astype(jnp.int32),
        ec0, ec1, t0p, t1p, zeros1, ones1)

    ngp = jnp.pad(node_graph.astype(jnp.int32), (0, NP - N),
                  constant_values=G)
    ng3 = ngp.reshape(NP // BF, 1, BF)
    n_out, g_out = pl.pallas_call(
        _finalize_body,
        grid=(NP // BF,),
        in_specs=[pl.BlockSpec((32, BF), lambda i: (0, i)),
                  pl.BlockSpec((12, BF), lambda i: (0, i)),
                  pl.BlockSpec((1, 1, BF), lambda i: (i, 0, 0)),
                  _full((32, 4)), _full((2, 4)), _full((1, 4)),
                  _full((4, 1)), _full((1, 1)),
                  _full((2, 1)), _full((4, 1)), _full((1, 1)),
                  _full((1, 1)), _full((1, 1)), _full((1, 1))],
        out_specs=[pl.BlockSpec((BF, 1), lambda i: (i, 0)),
                   pl.BlockSpec((G, 1), lambda i: (0, 0))],
        out_shape=[jax.ShapeDtypeStruct((N, 1), _f32),
                   jax.ShapeDtypeStruct((G, 1), _f32)],
        scratch_shapes=[pltpu.VMEM((G, 9), _f32)],
    )(nt, accf.reshape(12, NP), ng3,
      Wl_n_n, Wl_n_in, bln, Wr_n, br_n.reshape(1, 1),
      Wl_g_e, Wl_g_n, blg, Wr_g_g, Wr_g_n, br_g.reshape(1, 1))

    return (n_out, g_out)
